# Initial kernel scaffold; baseline (speedup 1.0000x reference)
import jax
import jax.numpy as jnp
from jax.experimental import pallas as pl


def kernel(x, W1, b1, W2, b2, edge_index):
    raise NotImplementedError("write your pallas kernel here")


# speedup vs baseline: 12.8503x; 12.8503x over previous
import functools

import jax
import jax.numpy as jnp
from jax import lax
from jax.experimental import pallas as pl
from jax.experimental.pallas import tpu as pltpu
from jax.experimental.pallas import tpu_sc as plsc

N = 10000
E = 320000
NC = 2
NS = 16
NW = NC * NS
CHUNK = 128
NCHUNK = 80
EPAD = NW * NCHUNK * CHUNK
PER_TILE = NCHUNK * CHUNK

NACC = 10240
ROWS_PER = NACC // NS

D1 = 67
D1P = 80
D2 = 40
D2P = 48


def _sc_degree(dst_hbm, zeros_hbm):
    mesh = plsc.VectorSubcoreMesh(core_axis_name="c", subcore_axis_name="s")

    @functools.partial(
        pl.kernel,
        out_type=jax.ShapeDtypeStruct((NC, NACC), jnp.float32),
        mesh=mesh,
        compiler_params=pltpu.CompilerParams(use_tc_tiling_on_sc=False),
        scratch_types=[
            pltpu.VMEM((NCHUNK, CHUNK), jnp.int32),
            pltpu.VMEM((CHUNK,), jnp.float32),
            pltpu.VMEM_SHARED((NACC,), jnp.float32),
        ],
    )
    def deg_kernel(dst_ref, zeros_ref, out_ref, dst_v, ones_v, acc_sh):
        cid = lax.axis_index("c")
        sid = lax.axis_index("s")
        wid = cid * NS + sid

        pltpu.sync_copy(zeros_ref.at[pl.ds(sid * ROWS_PER, ROWS_PER)],
                        acc_sh.at[pl.ds(sid * ROWS_PER, ROWS_PER)])
        pltpu.sync_copy(dst_ref.at[wid], dst_v)
        for i in range(CHUNK // 16):
            ones_v[pl.ds(16 * i, 16)] = jnp.ones((16,), jnp.float32)
        plsc.subcore_barrier()

        def body(j, _):
            pltpu.sync_copy(ones_v, acc_sh.at[dst_v.at[j]], add=True)
            return ()

        lax.fori_loop(0, NCHUNK, body, (), unroll=False)
        plsc.subcore_barrier()
        pltpu.sync_copy(acc_sh.at[pl.ds(sid * ROWS_PER, ROWS_PER)],
                        out_ref.at[cid].at[pl.ds(sid * ROWS_PER, ROWS_PER)])

    return deg_kernel(dst_hbm, zeros_hbm)


def _sc_aggregate(h_hbm, src_hbm, dst_hbm, zeros_hbm, d):
    mesh = plsc.VectorSubcoreMesh(core_axis_name="c", subcore_axis_name="s")

    @functools.partial(
        pl.kernel,
        out_type=jax.ShapeDtypeStruct((NC, NACC, d), jnp.float32),
        mesh=mesh,
        compiler_params=pltpu.CompilerParams(use_tc_tiling_on_sc=False),
        scratch_types=[
            pltpu.VMEM((NCHUNK, CHUNK), jnp.int32),
            pltpu.VMEM((NCHUNK, CHUNK), jnp.int32),
            pltpu.VMEM((CHUNK, d), jnp.float32),
            pltpu.VMEM_SHARED((NACC, d), jnp.float32),
            pltpu.SemaphoreType.DMA,
        ],
    )
    def agg_kernel(h_ref, src_ref, dst_ref, zeros_ref, out_ref,
                   src_v, dst_v, rows_v, acc_sh, sem):
        cid = lax.axis_index("c")
        sid = lax.axis_index("s")
        wid = cid * NS + sid

        pltpu.sync_copy(zeros_ref.at[pl.ds(sid * ROWS_PER, ROWS_PER)],
                        acc_sh.at[pl.ds(sid * ROWS_PER, ROWS_PER)])
        pltpu.sync_copy(src_ref.at[wid], src_v)
        pltpu.sync_copy(dst_ref.at[wid], dst_v)
        plsc.subcore_barrier()

        def body(j, _):
            pltpu.async_copy(h_ref.at[src_v.at[j]], rows_v, sem).wait()
            pltpu.sync_copy(rows_v, acc_sh.at[dst_v.at[j]], add=True)
            return ()

        lax.fori_loop(0, NCHUNK, body, (), unroll=False)
        plsc.subcore_barrier()
        pltpu.sync_copy(acc_sh.at[pl.ds(sid * ROWS_PER, ROWS_PER)],
                        out_ref.at[cid].at[pl.ds(sid * ROWS_PER, ROWS_PER)])

    return agg_kernel(h_hbm, src_hbm, dst_hbm, zeros_hbm)


def _mm_body(x_ref, w_ref, o_ref):
    o_ref[...] = jnp.dot(x_ref[...], w_ref[...],
                         preferred_element_type=jnp.float32,
                         precision=lax.Precision.HIGHEST)


def _tc_matmul(x, w):
    return pl.pallas_call(
        _mm_body,
        out_shape=jax.ShapeDtypeStruct((x.shape[0], w.shape[1]), jnp.float32),
    )(x, w)


def _dis_from(degs_ref):
    deg = degs_ref[0, :N] + degs_ref[1, :N] + 1.0
    return lax.rsqrt(deg)[:, None]


def _scale_body(h_ref, degs_ref, o_ref):
    o_ref[...] = h_ref[...] * _dis_from(degs_ref)


def _tc_scale(h, degs):
    return pl.pallas_call(
        _scale_body,
        out_shape=jax.ShapeDtypeStruct(h.shape, jnp.float32),
    )(h, degs)


def _combine_body(s_ref, h_ref, degs_ref, b_ref, w_ref, o_ref):
    dis = _dis_from(degs_ref)
    s = s_ref[0, :N, :] + s_ref[1, :N, :] + h_ref[...]
    t = dis * s + b_ref[...]
    t = jnp.where(t >= 0, t, 0.01 * t)
    o_ref[...] = jnp.dot(t, w_ref[...],
                         preferred_element_type=jnp.float32,
                         precision=lax.Precision.HIGHEST) * dis


def _tc_combine(s_parts, h, degs, b, w):
    return pl.pallas_call(
        _combine_body,
        out_shape=jax.ShapeDtypeStruct((N, w.shape[1]), jnp.float32),
    )(s_parts, h, degs, b, w)


def _final_body(s_ref, h_ref, degs_ref, b_ref, o_ref):
    dis = _dis_from(degs_ref)
    s = s_ref[0, :N, :] + s_ref[1, :N, :] + h_ref[...]
    t = dis * s + b_ref[...]
    valid = lax.broadcasted_iota(jnp.int32, (N, D2P), 1) < D2
    t = jnp.where(valid, t, -1e30)
    m = jnp.max(t, axis=1, keepdims=True)
    e = jnp.where(valid, jnp.exp(t - m), 0.0)
    se = jnp.sum(e, axis=1, keepdims=True)
    o_ref[...] = t - m - jnp.log(se)


def _tc_final(s_parts, h, degs, b):
    return pl.pallas_call(
        _final_body,
        out_shape=jax.ShapeDtypeStruct((N, D2P), jnp.float32),
    )(s_parts, h, degs, b)


def kernel(x, W1, b1, W2, b2, edge_index):
    src = edge_index[0].astype(jnp.int32)
    dst = edge_index[1].astype(jnp.int32)
    npad = EPAD - E
    src_p = jnp.concatenate([src, jnp.zeros((npad,), jnp.int32)])
    dst_p = jnp.concatenate([dst, jnp.full((npad,), N, jnp.int32)])
    src_p = src_p.reshape(NW, NCHUNK, CHUNK)
    dst_p = dst_p.reshape(NW, NCHUNK, CHUNK)

    zeros1 = jnp.zeros((NACC,), jnp.float32)
    zeros80 = jnp.zeros((NACC, D1P), jnp.float32)
    zeros48 = jnp.zeros((NACC, D2P), jnp.float32)

    W1p = jnp.pad(W1, ((0, 0), (0, D1P - D1)))
    b1p = jnp.pad(b1, (0, D1P - D1)).reshape(1, D1P)
    W2p = jnp.pad(W2, ((0, D1P - D1), (0, D2P - D2)))
    b2p = jnp.pad(b2, (0, D2P - D2)).reshape(1, D2P)

    degs = _sc_degree(dst_p, zeros1)
    h1 = _tc_matmul(x, W1p)
    h1p = _tc_scale(h1, degs)
    s1 = _sc_aggregate(h1p, src_p, dst_p, zeros80, D1P)
    h2p = _tc_combine(s1, h1p, degs, b1p, W2p)
    s2 = _sc_aggregate(h2p, src_p, dst_p, zeros48, D2P)
    out = _tc_final(s2, h2p, degs, b2p)
    return out[:, :D2]


# speedup vs baseline: 14.9096x; 1.1602x over previous
import functools

import jax
import jax.numpy as jnp
from jax import lax
from jax.experimental import pallas as pl
from jax.experimental.pallas import tpu as pltpu
from jax.experimental.pallas import tpu_sc as plsc

N = 10000
E = 320000
NC = 2
NS = 16
NW = NC * NS
CHUNK = 128
NCHUNK = 80
EPAD = NW * NCHUNK * CHUNK
PER_TILE = NCHUNK * CHUNK

NACC = 10240
ROWS_PER = NACC // NS

D1 = 67
D1P = 80
D2 = 40
D2P = 48


def _sc_degree(dst_hbm, zeros_hbm):
    mesh = plsc.VectorSubcoreMesh(core_axis_name="c", subcore_axis_name="s")

    @functools.partial(
        pl.kernel,
        out_type=jax.ShapeDtypeStruct((NC, NACC), jnp.float32),
        mesh=mesh,
        compiler_params=pltpu.CompilerParams(use_tc_tiling_on_sc=False),
        scratch_types=[
            pltpu.VMEM((NCHUNK, CHUNK), jnp.int32),
            pltpu.VMEM((CHUNK,), jnp.float32),
            pltpu.VMEM_SHARED((NACC,), jnp.float32),
        ],
    )
    def deg_kernel(dst_ref, zeros_ref, out_ref, dst_v, ones_v, acc_sh):
        cid = lax.axis_index("c")
        sid = lax.axis_index("s")
        wid = cid * NS + sid

        pltpu.sync_copy(zeros_ref.at[pl.ds(sid * ROWS_PER, ROWS_PER)],
                        acc_sh.at[pl.ds(sid * ROWS_PER, ROWS_PER)])
        pltpu.sync_copy(dst_ref.at[wid], dst_v)
        for i in range(CHUNK // 16):
            ones_v[pl.ds(16 * i, 16)] = jnp.ones((16,), jnp.float32)
        plsc.subcore_barrier()

        def body(j, _):
            pltpu.sync_copy(ones_v, acc_sh.at[dst_v.at[j]], add=True)
            return ()

        lax.fori_loop(0, NCHUNK, body, (), unroll=False)
        plsc.subcore_barrier()
        pltpu.sync_copy(acc_sh.at[pl.ds(sid * ROWS_PER, ROWS_PER)],
                        out_ref.at[cid].at[pl.ds(sid * ROWS_PER, ROWS_PER)])

    return deg_kernel(dst_hbm, zeros_hbm)


def _sc_aggregate(h_hbm, src_hbm, dst_hbm, zeros_hbm, d):
    mesh = plsc.VectorSubcoreMesh(core_axis_name="c", subcore_axis_name="s")

    @functools.partial(
        pl.kernel,
        out_type=jax.ShapeDtypeStruct((NC, NACC, d), jnp.float32),
        mesh=mesh,
        compiler_params=pltpu.CompilerParams(use_tc_tiling_on_sc=False),
        scratch_types=[
            pltpu.VMEM((NCHUNK, CHUNK), jnp.int32),
            pltpu.VMEM((NCHUNK, CHUNK), jnp.int32),
            pltpu.VMEM((CHUNK, d), jnp.float32),
            pltpu.VMEM((CHUNK, d), jnp.float32),
            pltpu.VMEM_SHARED((NACC, d), jnp.float32),
            pltpu.SemaphoreType.DMA,
            pltpu.SemaphoreType.DMA,
        ],
    )
    def agg_kernel(h_ref, src_ref, dst_ref, zeros_ref, out_ref,
                   src_v, dst_v, rows0_v, rows1_v, acc_sh, sem0, sem1):
        cid = lax.axis_index("c")
        sid = lax.axis_index("s")
        wid = cid * NS + sid

        pltpu.sync_copy(zeros_ref.at[pl.ds(sid * ROWS_PER, ROWS_PER)],
                        acc_sh.at[pl.ds(sid * ROWS_PER, ROWS_PER)])
        pltpu.sync_copy(src_ref.at[wid], src_v)
        pltpu.sync_copy(dst_ref.at[wid], dst_v)
        plsc.subcore_barrier()

        bufs = ((rows0_v, sem0), (rows1_v, sem1))
        for b in range(2):
            pltpu.async_copy(h_ref.at[src_v.at[b]], bufs[b][0], bufs[b][1])

        @pl.loop(0, NCHUNK, step=2)
        def _(j):
            for b in range(2):
                rows_v, sem = bufs[b]
                pltpu.make_async_copy(h_ref.at[src_v.at[j + b]],
                                      rows_v, sem).wait()
                pltpu.sync_copy(rows_v, acc_sh.at[dst_v.at[j + b]], add=True)

                @pl.when(j + b + 2 < NCHUNK)
                def _():
                    pltpu.async_copy(h_ref.at[src_v.at[j + b + 2]],
                                     rows_v, sem)
        plsc.subcore_barrier()
        pltpu.sync_copy(acc_sh.at[pl.ds(sid * ROWS_PER, ROWS_PER)],
                        out_ref.at[cid].at[pl.ds(sid * ROWS_PER, ROWS_PER)])

    return agg_kernel(h_hbm, src_hbm, dst_hbm, zeros_hbm)


def _mm_body(x_ref, w_ref, o_ref):
    o_ref[...] = jnp.dot(x_ref[...], w_ref[...],
                         preferred_element_type=jnp.float32,
                         precision=lax.Precision.HIGHEST)


def _tc_matmul(x, w):
    return pl.pallas_call(
        _mm_body,
        out_shape=jax.ShapeDtypeStruct((x.shape[0], w.shape[1]), jnp.float32),
    )(x, w)


def _dis_from(degs_ref):
    deg = degs_ref[0, :N] + degs_ref[1, :N] + 1.0
    return lax.rsqrt(deg)[:, None]


def _scale_body(h_ref, degs_ref, o_ref):
    o_ref[...] = h_ref[...] * _dis_from(degs_ref)


def _tc_scale(h, degs):
    return pl.pallas_call(
        _scale_body,
        out_shape=jax.ShapeDtypeStruct(h.shape, jnp.float32),
    )(h, degs)


def _combine_body(s_ref, h_ref, degs_ref, b_ref, w_ref, o_ref):
    dis = _dis_from(degs_ref)
    s = s_ref[0, :N, :] + s_ref[1, :N, :] + h_ref[...]
    t = dis * s + b_ref[...]
    t = jnp.where(t >= 0, t, 0.01 * t)
    o_ref[...] = jnp.dot(t, w_ref[...],
                         preferred_element_type=jnp.float32,
                         precision=lax.Precision.HIGHEST) * dis


def _tc_combine(s_parts, h, degs, b, w):
    return pl.pallas_call(
        _combine_body,
        out_shape=jax.ShapeDtypeStruct((N, w.shape[1]), jnp.float32),
    )(s_parts, h, degs, b, w)


def _final_body(s_ref, h_ref, degs_ref, b_ref, o_ref):
    dis = _dis_from(degs_ref)
    s = s_ref[0, :N, :] + s_ref[1, :N, :] + h_ref[...]
    t = dis * s + b_ref[...]
    valid = lax.broadcasted_iota(jnp.int32, (N, D2P), 1) < D2
    t = jnp.where(valid, t, -1e30)
    m = jnp.max(t, axis=1, keepdims=True)
    e = jnp.where(valid, jnp.exp(t - m), 0.0)
    se = jnp.sum(e, axis=1, keepdims=True)
    o_ref[...] = t - m - jnp.log(se)


def _tc_final(s_parts, h, degs, b):
    return pl.pallas_call(
        _final_body,
        out_shape=jax.ShapeDtypeStruct((N, D2P), jnp.float32),
    )(s_parts, h, degs, b)


def kernel(x, W1, b1, W2, b2, edge_index):
    src = edge_index[0].astype(jnp.int32)
    dst = edge_index[1].astype(jnp.int32)
    npad = EPAD - E
    src_p = jnp.concatenate([src, jnp.zeros((npad,), jnp.int32)])
    dst_p = jnp.concatenate([dst, jnp.full((npad,), N, jnp.int32)])
    src_p = src_p.reshape(NW, NCHUNK, CHUNK)
    dst_p = dst_p.reshape(NW, NCHUNK, CHUNK)

    zeros1 = jnp.zeros((NACC,), jnp.float32)
    zeros80 = jnp.zeros((NACC, D1P), jnp.float32)
    zeros48 = jnp.zeros((NACC, D2P), jnp.float32)

    W1p = jnp.pad(W1, ((0, 0), (0, D1P - D1)))
    b1p = jnp.pad(b1, (0, D1P - D1)).reshape(1, D1P)
    W2p = jnp.pad(W2, ((0, D1P - D1), (0, D2P - D2)))
    b2p = jnp.pad(b2, (0, D2P - D2)).reshape(1, D2P)

    degs = _sc_degree(dst_p, zeros1)
    h1 = _tc_matmul(x, W1p)
    h1p = _tc_scale(h1, degs)
    s1 = _sc_aggregate(h1p, src_p, dst_p, zeros80, D1P)
    h2p = _tc_combine(s1, h1p, degs, b1p, W2p)
    s2 = _sc_aggregate(h2p, src_p, dst_p, zeros48, D2P)
    out = _tc_final(s2, h2p, degs, b2p)
    return out[:, :D2]


# speedup vs baseline: 32.7116x; 2.1940x over previous
import functools

import jax
import jax.numpy as jnp
from jax import lax
from jax.experimental import pallas as pl
from jax.experimental.pallas import tpu as pltpu
from jax.experimental.pallas import tpu_sc as plsc

N = 10000
E = 320000
NC = 2
NS = 16
NW = NC * NS
CHUNK = 128
NCHUNK = 80
EPAD = NW * NCHUNK * CHUNK
PER_TILE = NCHUNK * CHUNK

NACC = 10240
ROWS_PER = NACC // NS

D1 = 67
D1P = 80
DHALF = D1P // 2
D2 = 40
D2P = 48


def _sc_degree(dst_hbm, zeros_hbm):
    mesh = plsc.VectorSubcoreMesh(core_axis_name="c", subcore_axis_name="s")

    @functools.partial(
        pl.kernel,
        out_type=jax.ShapeDtypeStruct((NC, NACC), jnp.float32),
        mesh=mesh,
        compiler_params=pltpu.CompilerParams(use_tc_tiling_on_sc=False),
        scratch_types=[
            pltpu.VMEM((NCHUNK, CHUNK), jnp.int32),
            pltpu.VMEM((CHUNK,), jnp.float32),
            pltpu.VMEM_SHARED((NACC,), jnp.float32),
        ],
    )
    def deg_kernel(dst_ref, zeros_ref, out_ref, dst_v, ones_v, acc_sh):
        cid = lax.axis_index("c")
        sid = lax.axis_index("s")
        wid = cid * NS + sid

        pltpu.sync_copy(zeros_ref.at[pl.ds(sid * ROWS_PER, ROWS_PER)],
                        acc_sh.at[pl.ds(sid * ROWS_PER, ROWS_PER)])
        pltpu.sync_copy(dst_ref.at[wid], dst_v)
        for i in range(CHUNK // 16):
            ones_v[pl.ds(16 * i, 16)] = jnp.ones((16,), jnp.float32)
        plsc.subcore_barrier()

        def body(j, _):
            pltpu.sync_copy(ones_v, acc_sh.at[dst_v.at[j]], add=True)
            return ()

        lax.fori_loop(0, NCHUNK, body, (), unroll=False)
        plsc.subcore_barrier()
        pltpu.sync_copy(acc_sh.at[pl.ds(sid * ROWS_PER, ROWS_PER)],
                        out_ref.at[cid].at[pl.ds(sid * ROWS_PER, ROWS_PER)])

    return deg_kernel(dst_hbm, zeros_hbm)


def _sc_aggregate(h_hbm, src_hbm, dst_hbm, zeros_hbm, d):
    mesh = plsc.VectorSubcoreMesh(core_axis_name="c", subcore_axis_name="s")

    @functools.partial(
        pl.kernel,
        out_type=jax.ShapeDtypeStruct((NC, NACC, d), jnp.float32),
        mesh=mesh,
        compiler_params=pltpu.CompilerParams(use_tc_tiling_on_sc=False),
        scratch_types=[
            pltpu.VMEM((NCHUNK, CHUNK), jnp.int32),
            pltpu.VMEM((NCHUNK, CHUNK), jnp.int32),
            pltpu.VMEM((CHUNK, d), jnp.float32),
            pltpu.VMEM((CHUNK, d), jnp.float32),
            pltpu.VMEM_SHARED((NACC, d), jnp.float32),
            pltpu.VMEM_SHARED((NACC, d), jnp.float32),
            pltpu.SemaphoreType.DMA,
            pltpu.SemaphoreType.DMA,
        ],
    )
    def agg_kernel(h_ref, src_ref, dst_ref, zeros_ref, out_ref,
                   src_v, dst_v, rows0_v, rows1_v, tbl_sh, acc_sh,
                   sem0, sem1):
        cid = lax.axis_index("c")
        sid = lax.axis_index("s")
        wid = cid * NS + sid

        sl = pl.ds(sid * ROWS_PER, ROWS_PER)
        pltpu.sync_copy(h_ref.at[sl], tbl_sh.at[sl])
        pltpu.sync_copy(zeros_ref.at[sl], acc_sh.at[sl])
        pltpu.sync_copy(src_ref.at[wid], src_v)
        pltpu.sync_copy(dst_ref.at[wid], dst_v)
        plsc.subcore_barrier()

        bufs = ((rows0_v, sem0), (rows1_v, sem1))
        for b in range(2):
            pltpu.async_copy(tbl_sh.at[src_v.at[b]], bufs[b][0], bufs[b][1])

        @pl.loop(0, NCHUNK, step=2)
        def _(j):
            for b in range(2):
                rows_v, sem = bufs[b]
                pltpu.make_async_copy(tbl_sh.at[src_v.at[j + b]],
                                      rows_v, sem).wait()
                pltpu.sync_copy(rows_v, acc_sh.at[dst_v.at[j + b]], add=True)

                @pl.when(j + b + 2 < NCHUNK)
                def _():
                    pltpu.async_copy(tbl_sh.at[src_v.at[j + b + 2]],
                                     rows_v, sem)

        plsc.subcore_barrier()
        pltpu.sync_copy(acc_sh.at[sl], out_ref.at[cid].at[sl])

    return agg_kernel(h_hbm, src_hbm, dst_hbm, zeros_hbm)


def _mm_body(x_ref, w_ref, o_ref):
    o_ref[...] = jnp.dot(x_ref[...], w_ref[...],
                         preferred_element_type=jnp.float32,
                         precision=lax.Precision.HIGHEST)


def _tc_matmul(x, w):
    return pl.pallas_call(
        _mm_body,
        out_shape=jax.ShapeDtypeStruct((x.shape[0], w.shape[1]), jnp.float32),
    )(x, w)


def _dis_from(degs_ref):
    deg = degs_ref[0, :] + degs_ref[1, :] + 1.0
    return lax.rsqrt(deg)[:, None]


def _scale_body(h_ref, degs_ref, o_ref):
    o_ref[...] = h_ref[...] * _dis_from(degs_ref)


def _tc_scale(h, degs):
    return pl.pallas_call(
        _scale_body,
        out_shape=jax.ShapeDtypeStruct(h.shape, jnp.float32),
    )(h, degs)


def _combine_body(sa_ref, sb_ref, h_ref, degs_ref, b_ref, wa_ref, wb_ref,
                  o_ref):
    dis = _dis_from(degs_ref)
    ha = h_ref[:, :DHALF]
    hb = h_ref[:, DHALF:]
    ta = dis * (sa_ref[0] + sa_ref[1] + ha) + b_ref[:, :DHALF]
    tb = dis * (sb_ref[0] + sb_ref[1] + hb) + b_ref[:, DHALF:]
    ta = jnp.where(ta >= 0, ta, 0.01 * ta)
    tb = jnp.where(tb >= 0, tb, 0.01 * tb)
    o_ref[...] = (jnp.dot(ta, wa_ref[...],
                          preferred_element_type=jnp.float32,
                          precision=lax.Precision.HIGHEST)
                  + jnp.dot(tb, wb_ref[...],
                            preferred_element_type=jnp.float32,
                            precision=lax.Precision.HIGHEST)) * dis


RB = 2048


def _tc_combine(sa, sb, h, degs, b, wa, wb):
    return pl.pallas_call(
        _combine_body,
        grid=(NACC // RB,),
        in_specs=[
            pl.BlockSpec((2, RB, DHALF), lambda i: (0, i, 0)),
            pl.BlockSpec((2, RB, DHALF), lambda i: (0, i, 0)),
            pl.BlockSpec((RB, D1P), lambda i: (i, 0)),
            pl.BlockSpec((2, RB), lambda i: (0, i)),
            pl.BlockSpec((1, D1P), lambda i: (0, 0)),
            pl.BlockSpec((DHALF, D2P), lambda i: (0, 0)),
            pl.BlockSpec((DHALF, D2P), lambda i: (0, 0)),
        ],
        out_specs=pl.BlockSpec((RB, D2P), lambda i: (i, 0)),
        out_shape=jax.ShapeDtypeStruct((NACC, wa.shape[1]), jnp.float32),
    )(sa, sb, h, degs, b, wa, wb)


def _final_body(s_ref, h_ref, degs_ref, b_ref, o_ref):
    dis = _dis_from(degs_ref)
    s = s_ref[0] + s_ref[1] + h_ref[...]
    t = dis * s + b_ref[...]
    valid = lax.broadcasted_iota(jnp.int32, (RB, D2P), 1) < D2
    t = jnp.where(valid, t, -1e30)
    m = jnp.max(t, axis=1, keepdims=True)
    e = jnp.where(valid, jnp.exp(t - m), 0.0)
    se = jnp.sum(e, axis=1, keepdims=True)
    o_ref[...] = t - m - jnp.log(se)


def _tc_final(s_parts, h, degs, b):
    return pl.pallas_call(
        _final_body,
        grid=(NACC // RB,),
        in_specs=[
            pl.BlockSpec((2, RB, D2P), lambda i: (0, i, 0)),
            pl.BlockSpec((RB, D2P), lambda i: (i, 0)),
            pl.BlockSpec((2, RB), lambda i: (0, i)),
            pl.BlockSpec((1, D2P), lambda i: (0, 0)),
        ],
        out_specs=pl.BlockSpec((RB, D2P), lambda i: (i, 0)),
        out_shape=jax.ShapeDtypeStruct((NACC, D2P), jnp.float32),
    )(s_parts, h, degs, b)


def kernel(x, W1, b1, W2, b2, edge_index):
    src = edge_index[0].astype(jnp.int32)
    dst = edge_index[1].astype(jnp.int32)
    npad = EPAD - E
    src_p = jnp.concatenate([src, jnp.zeros((npad,), jnp.int32)])
    dst_p = jnp.concatenate([dst, jnp.full((npad,), N, jnp.int32)])
    src_p = src_p.reshape(NW, NCHUNK, CHUNK)
    dst_p = dst_p.reshape(NW, NCHUNK, CHUNK)

    xp = jnp.pad(x, ((0, NACC - N), (0, 0)))
    zeros1 = jnp.zeros((NACC,), jnp.float32)
    zeros40 = jnp.zeros((NACC, DHALF), jnp.float32)
    zeros48 = jnp.zeros((NACC, D2P), jnp.float32)

    W1p = jnp.pad(W1, ((0, 0), (0, D1P - D1)))
    b1p = jnp.pad(b1, (0, D1P - D1)).reshape(1, D1P)
    W2p = jnp.pad(W2, ((0, D1P - D1), (0, D2P - D2)))
    b2p = jnp.pad(b2, (0, D2P - D2)).reshape(1, D2P)

    degs = _sc_degree(dst_p, zeros1)
    h1 = _tc_matmul(xp, W1p)
    h1p = _tc_scale(h1, degs)
    h1pa = jnp.asarray(h1p[:, :DHALF])
    h1pb = jnp.asarray(h1p[:, DHALF:])
    s1a = _sc_aggregate(h1pa, src_p, dst_p, zeros40, DHALF)
    s1b = _sc_aggregate(h1pb, src_p, dst_p, zeros40, DHALF)
    h2p = _tc_combine(s1a, s1b, h1p, degs, b1p,
                      W2p[:DHALF], W2p[DHALF:])
    s2 = _sc_aggregate(h2p, src_p, dst_p, zeros48, D2P)
    out = _tc_final(s2, h2p, degs, b2p)
    return out[:N, :D2]
